# Initial kernel scaffold; baseline (speedup 1.0000x reference)
import jax
import jax.numpy as jnp
from jax.experimental import pallas as pl


def kernel(position_ids, embedding_table):
    raise NotImplementedError("write your pallas kernel here")


# speedup vs baseline: 2.2501x; 2.2501x over previous
import functools

import jax
import jax.numpy as jnp
from jax import lax
from jax.experimental import pallas as pl
from jax.experimental.pallas import tpu as pltpu
from jax.experimental.pallas import tpu_sc as plsc

_NUM_CORES = 2
_NUM_SUBCORES = 16
_NW = _NUM_CORES * _NUM_SUBCORES

_CHUNK = 32


@functools.lru_cache(maxsize=None)
def _make_gather(total: int, hidden: int):
    assert total % (_NW * 2 * _CHUNK) == 0
    b_per_w = total // _NW
    n_chunks = b_per_w // _CHUNK
    n_pairs = n_chunks // 2

    mesh = plsc.VectorSubcoreMesh(core_axis_name="c", subcore_axis_name="s")

    @functools.partial(
        pl.kernel,
        mesh=mesh,
        out_type=jax.ShapeDtypeStruct((total, hidden), jnp.float32),
        scratch_types=[
            pltpu.VMEM((n_chunks, _CHUNK), jnp.int32),
            pltpu.VMEM((_CHUNK, hidden), jnp.float32),
            pltpu.VMEM((_CHUNK, hidden), jnp.float32),
            pltpu.SemaphoreType.DMA,
            pltpu.SemaphoreType.DMA,
            pltpu.SemaphoreType.DMA,
            pltpu.SemaphoreType.DMA,
        ],
    )
    def gather_kernel(idx_hbm, table_hbm, out_hbm, idx_v, buf0, buf1,
                      sg0, sg1, so0, so1):
        wid = lax.axis_index("s") * _NUM_CORES + lax.axis_index("c")
        base = wid * b_per_w

        pltpu.sync_copy(idx_hbm.at[wid], idx_v)

        def gather_start(c, buf, sem):
            pltpu.async_copy(table_hbm.at[idx_v.at[c]], buf, sem)

        def gather_wait(c, buf, sem):
            pltpu.make_async_copy(table_hbm.at[idx_v.at[c]], buf, sem).wait()

        def out_start(c, buf, sem):
            pltpu.async_copy(buf, out_hbm.at[pl.ds(base + c * _CHUNK, _CHUNK)],
                             sem)

        def out_wait(buf, sem):
            pltpu.make_async_copy(buf, out_hbm.at[pl.ds(base, _CHUNK)],
                                  sem).wait()

        gather_start(0, buf0, sg0)

        def pair_body(p, carry):
            c0 = 2 * p
            c1 = c0 + 1

            @pl.when(p > 0)
            def _():
                out_wait(buf1, so1)

            gather_start(c1, buf1, sg1)

            gather_wait(c0, buf0, sg0)
            out_start(c0, buf0, so0)

            gather_wait(c1, buf1, sg1)
            out_start(c1, buf1, so1)

            @pl.when(p + 1 < n_pairs)
            def _():
                out_wait(buf0, so0)
                gather_start(c0 + 2, buf0, sg0)

            return carry

        lax.fori_loop(0, n_pairs, pair_body, 0)

        out_wait(buf0, so0)
        out_wait(buf1, so1)

    return gather_kernel


def kernel(position_ids, embedding_table):
    batch, seq = position_ids.shape
    _, hidden = embedding_table.shape
    total = batch * seq

    b_per_w = total // _NW
    n_chunks = b_per_w // _CHUNK
    ids = position_ids.astype(jnp.int32).reshape(_NW, n_chunks, _CHUNK)
    table = embedding_table.astype(jnp.float32)

    out = _make_gather(total, hidden)(ids, table)
    return out.reshape(batch, seq, hidden)


# speedup vs baseline: 2.3795x; 1.0575x over previous
import functools

import jax
import jax.numpy as jnp
from jax import lax
from jax.experimental import pallas as pl
from jax.experimental.pallas import tpu as pltpu
from jax.experimental.pallas import tpu_sc as plsc

_NUM_CORES = 2
_NUM_SUBCORES = 16
_NW = _NUM_CORES * _NUM_SUBCORES

_CHUNK = 16
_NBUF = 4


@functools.lru_cache(maxsize=None)
def _make_gather(total: int, hidden: int):
    assert total % (_NW * _NBUF * _CHUNK) == 0
    b_per_w = total // _NW
    n_chunks = b_per_w // _CHUNK
    n_groups = n_chunks // _NBUF

    mesh = plsc.VectorSubcoreMesh(core_axis_name="c", subcore_axis_name="s")

    scratch = [pltpu.VMEM((n_chunks, _CHUNK), jnp.int32)]
    scratch += [pltpu.VMEM((_CHUNK, hidden), jnp.float32)
                for _ in range(_NBUF)]
    scratch += [pltpu.SemaphoreType.DMA for _ in range(2 * _NBUF)]

    @functools.partial(
        pl.kernel,
        mesh=mesh,
        out_type=jax.ShapeDtypeStruct((total, hidden), jnp.float32),
        scratch_types=scratch,
    )
    def gather_kernel(idx_hbm, table_hbm, out_hbm, idx_v, *rest):
        bufs = rest[:_NBUF]
        sg = rest[_NBUF:2 * _NBUF]
        so = rest[2 * _NBUF:]

        wid = lax.axis_index("s") * _NUM_CORES + lax.axis_index("c")
        base = wid * b_per_w

        pltpu.sync_copy(idx_hbm.at[wid], idx_v)

        def gather_start(c, buf, sem):
            pltpu.async_copy(table_hbm.at[idx_v.at[c]], buf, sem)

        def gather_wait(c, buf, sem):
            pltpu.make_async_copy(table_hbm.at[idx_v.at[c]], buf, sem).wait()

        def out_start(c, buf, sem):
            pltpu.async_copy(buf, out_hbm.at[pl.ds(base + c * _CHUNK, _CHUNK)],
                             sem)

        def out_wait(buf, sem):
            pltpu.make_async_copy(buf, out_hbm.at[pl.ds(base, _CHUNK)],
                                  sem).wait()

        for b in range(_NBUF):
            gather_start(b, bufs[b], sg[b])

        def group_body(g, carry):
            c0 = g * _NBUF
            for b in range(_NBUF):
                gather_wait(c0 + b, bufs[b], sg[b])
                out_start(c0 + b, bufs[b], so[b])

                @pl.when(g + 1 < n_groups)
                def _(b=b, c0=c0):
                    out_wait(bufs[b], so[b])
                    gather_start(c0 + _NBUF + b, bufs[b], sg[b])

            return carry

        lax.fori_loop(0, n_groups, group_body, 0)

        for b in range(_NBUF):
            out_wait(bufs[b], so[b])

    return gather_kernel


def kernel(position_ids, embedding_table):
    batch, seq = position_ids.shape
    _, hidden = embedding_table.shape
    total = batch * seq

    b_per_w = total // _NW
    n_chunks = b_per_w // _CHUNK
    ids = position_ids.astype(jnp.int32).reshape(_NW, n_chunks, _CHUNK)
    table = embedding_table.astype(jnp.float32)

    out = _make_gather(total, hidden)(ids, table)
    return out.reshape(batch, seq, hidden)
